# baseline (speedup 1.0000x reference)
import functools

import jax
import jax.numpy as jnp
from jax import lax
from jax.experimental import pallas as pl
from jax.experimental.pallas import tpu as pltpu
from jax.experimental.pallas import tpu_sc as plsc

NC = 2
NS = 16
NW = NC * NS
D = 64

DN = 16


def _gather_kernel(N: int, M: int):
    n_per_w = N // NW
    n_chunks = n_per_w // DN
    assert n_chunks % 2 == 0
    K = n_chunks // 2
    mesh = plsc.VectorSubcoreMesh(core_axis_name="c", subcore_axis_name="s")

    @functools.partial(
        pl.kernel,
        mesh=mesh,
        out_type=jax.ShapeDtypeStruct((N, M, D), jnp.float32),
        scratch_types=[
            pltpu.VMEM((DN, M), jnp.int32),
            pltpu.VMEM((DN, M), jnp.int32),
            pltpu.VMEM((DN, M, D), jnp.float32),
            pltpu.VMEM((DN, M, D), jnp.float32),
            pltpu.SemaphoreType.DMA,
            pltpu.SemaphoreType.DMA,
            pltpu.SemaphoreType.DMA,
            pltpu.SemaphoreType.DMA,
        ],
        compiler_params=pltpu.CompilerParams(use_tc_tiling_on_sc=False),
    )
    def k(idx_hbm, table_hbm, out_hbm, idx0, idx1, rows0, rows1,
          gsem0, gsem1, wsem0, wsem1):
        wid = lax.axis_index("s") * NC + lax.axis_index("c")
        base = wid * n_per_w

        def fire_gathers(idx_v, rows_v, sem):
            for i in range(DN):
                pltpu.async_copy(
                    table_hbm.at[idx_v.at[i]], rows_v.at[i], sem)

        def drain_gathers(idx_v, rows_v, sem):
            for i in range(DN):
                pltpu.make_async_copy(
                    table_hbm.at[idx_v.at[i]], rows_v.at[i], sem).wait()

        def drain_write(rows_v, sem):
            pltpu.make_async_copy(
                rows_v, out_hbm.at[pl.ds(0, DN)], sem).wait()

        pltpu.sync_copy(idx_hbm.at[pl.ds(base, DN)], idx0)
        fire_gathers(idx0, rows0, gsem0)

        def body(kk, carry):
            o0 = base + (2 * kk) * DN
            o1 = o0 + DN
            o2 = o0 + 2 * DN
            pltpu.sync_copy(idx_hbm.at[pl.ds(o1, DN)], idx1)

            @pl.when(kk > 0)
            def _():
                drain_write(rows1, wsem1)

            fire_gathers(idx1, rows1, gsem1)
            drain_gathers(idx0, rows0, gsem0)
            pltpu.async_copy(rows0, out_hbm.at[pl.ds(o0, DN)], wsem0)

            @pl.when(kk < K - 1)
            def _():
                pltpu.sync_copy(idx_hbm.at[pl.ds(o2, DN)], idx0)
                drain_write(rows0, wsem0)
                fire_gathers(idx0, rows0, gsem0)

            drain_gathers(idx1, rows1, gsem1)
            pltpu.async_copy(rows1, out_hbm.at[pl.ds(o1, DN)], wsem1)
            return carry

        lax.fori_loop(0, K, body, 0)
        drain_write(rows0, wsem0)
        drain_write(rows1, wsem1)

    return k


def kernel(x, embedding):
    n, m = x.shape
    return _gather_kernel(n, m)(x.astype(jnp.int32), embedding)
